# baseline (speedup 1.0000x reference)
import jax
import jax.numpy as jnp
from jax import lax
from jax.experimental import pallas as pl
from jax.experimental.pallas import tpu as pltpu

_LANE = 128
_UNROLL = 128
_NSEM = 1


def _round_up(a: int, b: int) -> int:
    return (a + b - 1) // b * b


def _gather_kernel(ids_ref, table_ref, out_ref, sem):
    i = pl.program_id(0)
    t_block = out_ref.shape[0]
    base = i * t_block

    def issue(jo, carry):
        for u in range(_UNROLL):
            j = jo * _UNROLL + u
            idx = ids_ref[base + j]
            pltpu.make_async_copy(
                table_ref.at[pl.ds(idx, 1), :],
                out_ref.at[pl.ds(j, 1), :],
                sem.at[u % _NSEM],
            ).start()
        return carry

    lax.fori_loop(0, t_block // _UNROLL, issue, 0)

    for q in range(_NSEM):
        pltpu.make_async_copy(
            table_ref.at[pl.ds(0, t_block // _NSEM)],
            out_ref.at[pl.ds(0, t_block // _NSEM)],
            sem.at[q],
        ).wait()


def token_embedding_gather(x, table, *, block_tokens=2048, interpret=False):
    batch, seq = x.shape
    vocab_size, d_model = table.shape
    assert d_model % _LANE == 0, "d_model must be a multiple of 128"
    n_tokens = batch * seq

    ids = jnp.clip(x.reshape(-1), 0, vocab_size - 1).astype(jnp.int32)

    t_block = min(block_tokens, _round_up(n_tokens, _UNROLL))
    t_block = _round_up(t_block, _UNROLL)
    n_pad = _round_up(n_tokens, t_block)
    if n_pad != n_tokens:
        ids = jnp.pad(ids, (0, n_pad - n_tokens))
    num_blocks = n_pad // t_block

    out = pl.pallas_call(
        _gather_kernel,
        out_shape=jax.ShapeDtypeStruct((n_pad, d_model), table.dtype),
        grid=(num_blocks,),
        in_specs=[
            pl.BlockSpec(memory_space=pltpu.SMEM),
            pl.BlockSpec(memory_space=pl.ANY),
        ],
        out_specs=pl.BlockSpec((t_block, d_model), lambda i: (i, 0)),
        scratch_shapes=[pltpu.SemaphoreType.DMA((_NSEM,))],
        compiler_params=pltpu.CompilerParams(
            dimension_semantics=("parallel",),
            disable_bounds_checks=True,
            vmem_limit_bytes=64 << 20,
        ),
        interpret=interpret,
    )(ids, table)

    if n_pad != n_tokens:
        out = out[:n_tokens]
    return out.reshape(batch, seq, d_model)


def kernel(x, table):
    return token_embedding_gather(x, table)


# speedup vs baseline: 1.0188x; 1.0188x over previous
import jax
import jax.numpy as jnp
from jax import lax
from jax.experimental import pallas as pl
from jax.experimental.pallas import tpu as pltpu

_LANE = 128
_UNROLL = 256
_NSEM = 1


def _round_up(a: int, b: int) -> int:
    return (a + b - 1) // b * b


def _gather_kernel(ids_ref, table_ref, out_ref, sem):
    i = pl.program_id(0)
    t_block = out_ref.shape[0]
    base = i * t_block

    def issue(jo, carry):
        for u in range(_UNROLL):
            j = jo * _UNROLL + u
            idx = ids_ref[base + j]
            pltpu.make_async_copy(
                table_ref.at[pl.ds(idx, 1), :],
                out_ref.at[pl.ds(j, 1), :],
                sem.at[u % _NSEM],
            ).start()
        return carry

    lax.fori_loop(0, t_block // _UNROLL, issue, 0)

    for q in range(_NSEM):
        pltpu.make_async_copy(
            table_ref.at[pl.ds(0, t_block // _NSEM)],
            out_ref.at[pl.ds(0, t_block // _NSEM)],
            sem.at[q],
        ).wait()


def token_embedding_gather(x, table, *, block_tokens=4096, interpret=False):
    batch, seq = x.shape
    vocab_size, d_model = table.shape
    assert d_model % _LANE == 0, "d_model must be a multiple of 128"
    n_tokens = batch * seq

    ids = jnp.clip(x.reshape(-1), 0, vocab_size - 1).astype(jnp.int32)

    t_block = min(block_tokens, _round_up(n_tokens, _UNROLL))
    t_block = _round_up(t_block, _UNROLL)
    n_pad = _round_up(n_tokens, t_block)
    if n_pad != n_tokens:
        ids = jnp.pad(ids, (0, n_pad - n_tokens))
    num_blocks = n_pad // t_block

    out = pl.pallas_call(
        _gather_kernel,
        out_shape=jax.ShapeDtypeStruct((n_pad, d_model), table.dtype),
        grid=(num_blocks,),
        in_specs=[
            pl.BlockSpec(memory_space=pltpu.SMEM),
            pl.BlockSpec(memory_space=pl.ANY),
        ],
        out_specs=pl.BlockSpec((t_block, d_model), lambda i: (i, 0)),
        scratch_shapes=[pltpu.SemaphoreType.DMA((_NSEM,))],
        compiler_params=pltpu.CompilerParams(
            dimension_semantics=("parallel",),
            disable_bounds_checks=True,
            vmem_limit_bytes=64 << 20,
        ),
        interpret=interpret,
    )(ids, table)

    if n_pad != n_tokens:
        out = out[:n_tokens]
    return out.reshape(batch, seq, d_model)


def kernel(x, table):
    return token_embedding_gather(x, table)


# speedup vs baseline: 1.0206x; 1.0018x over previous
import jax
import jax.numpy as jnp
from jax import lax
from jax.experimental import pallas as pl
from jax.experimental.pallas import tpu as pltpu

_LANE = 128
_UNROLL = 512
_NSEM = 1


def _round_up(a: int, b: int) -> int:
    return (a + b - 1) // b * b


def _gather_kernel(ids_ref, table_ref, out_ref, sem):
    i = pl.program_id(0)
    t_block = out_ref.shape[0]
    base = i * t_block

    def issue(jo, carry):
        for u in range(_UNROLL):
            j = jo * _UNROLL + u
            idx = ids_ref[base + j]
            pltpu.make_async_copy(
                table_ref.at[pl.ds(idx, 1), :],
                out_ref.at[pl.ds(j, 1), :],
                sem.at[u % _NSEM],
            ).start()
        return carry

    lax.fori_loop(0, t_block // _UNROLL, issue, 0)

    for q in range(_NSEM):
        pltpu.make_async_copy(
            table_ref.at[pl.ds(0, t_block // _NSEM)],
            out_ref.at[pl.ds(0, t_block // _NSEM)],
            sem.at[q],
        ).wait()


def token_embedding_gather(x, table, *, block_tokens=4096, interpret=False):
    batch, seq = x.shape
    vocab_size, d_model = table.shape
    assert d_model % _LANE == 0, "d_model must be a multiple of 128"
    n_tokens = batch * seq

    ids = jnp.clip(x.reshape(-1), 0, vocab_size - 1).astype(jnp.int32)

    t_block = min(block_tokens, _round_up(n_tokens, _UNROLL))
    t_block = _round_up(t_block, _UNROLL)
    n_pad = _round_up(n_tokens, t_block)
    if n_pad != n_tokens:
        ids = jnp.pad(ids, (0, n_pad - n_tokens))
    num_blocks = n_pad // t_block

    out = pl.pallas_call(
        _gather_kernel,
        out_shape=jax.ShapeDtypeStruct((n_pad, d_model), table.dtype),
        grid=(num_blocks,),
        in_specs=[
            pl.BlockSpec(memory_space=pltpu.SMEM),
            pl.BlockSpec(memory_space=pl.ANY),
        ],
        out_specs=pl.BlockSpec((t_block, d_model), lambda i: (i, 0)),
        scratch_shapes=[pltpu.SemaphoreType.DMA((_NSEM,))],
        compiler_params=pltpu.CompilerParams(
            dimension_semantics=("parallel",),
            disable_bounds_checks=True,
            vmem_limit_bytes=64 << 20,
        ),
        interpret=interpret,
    )(ids, table)

    if n_pad != n_tokens:
        out = out[:n_tokens]
    return out.reshape(batch, seq, d_model)


def kernel(x, table):
    return token_embedding_gather(x, table)
